# Initial kernel scaffold; baseline (speedup 1.0000x reference)
import jax
import jax.numpy as jnp
from jax.experimental import pallas as pl


def kernel(x, pos_table):
    raise NotImplementedError("write your pallas kernel here")


# speedup vs baseline: 1.0131x; 1.0131x over previous
import jax
import jax.numpy as jnp
from jax.experimental import pallas as pl

NUM_PATCHES = 1024
D_MODEL = 768
BATCH = 64


def _add_kernel(x_ref, pos_ref, o_ref):
    o_ref[...] = x_ref[...] + pos_ref[...]


def kernel(x, pos_table):
    grid = (BATCH,)
    return pl.pallas_call(
        _add_kernel,
        grid=grid,
        in_specs=[
            pl.BlockSpec((1, NUM_PATCHES, D_MODEL), lambda b: (b, 0, 0)),
            pl.BlockSpec((NUM_PATCHES, D_MODEL), lambda b: (0, 0)),
        ],
        out_specs=pl.BlockSpec((1, NUM_PATCHES, D_MODEL), lambda b: (b, 0, 0)),
        out_shape=jax.ShapeDtypeStruct((BATCH, NUM_PATCHES, D_MODEL), x.dtype),
    )(x, pos_table)


# speedup vs baseline: 1.0466x; 1.0331x over previous
import jax
import jax.numpy as jnp
from jax.experimental import pallas as pl

NUM_PATCHES = 1024
D_MODEL = 768
BATCH = 64


def _add_kernel(x_ref, pos_ref, o_ref):
    o_ref[...] = x_ref[...] + pos_ref[...]


def kernel(x, pos_table):
    bb = 2
    grid = (BATCH // bb,)
    return pl.pallas_call(
        _add_kernel,
        grid=grid,
        in_specs=[
            pl.BlockSpec((bb, NUM_PATCHES, D_MODEL), lambda b: (b, 0, 0)),
            pl.BlockSpec((NUM_PATCHES, D_MODEL), lambda b: (0, 0)),
        ],
        out_specs=pl.BlockSpec((bb, NUM_PATCHES, D_MODEL), lambda b: (b, 0, 0)),
        out_shape=jax.ShapeDtypeStruct((BATCH, NUM_PATCHES, D_MODEL), x.dtype),
    )(x, pos_table)


# speedup vs baseline: 1.0564x; 1.0093x over previous
import jax
import jax.numpy as jnp
from jax.experimental import pallas as pl

NUM_PATCHES = 1024
D_MODEL = 768
BATCH = 64


def _add_kernel(x_ref, pos_ref, o_ref):
    o_ref[...] = x_ref[...] + pos_ref[...]


def kernel(x, pos_table):
    bb = 4
    grid = (BATCH // bb,)
    return pl.pallas_call(
        _add_kernel,
        grid=grid,
        in_specs=[
            pl.BlockSpec((bb, NUM_PATCHES, D_MODEL), lambda b: (b, 0, 0)),
            pl.BlockSpec((NUM_PATCHES, D_MODEL), lambda b: (0, 0)),
        ],
        out_specs=pl.BlockSpec((bb, NUM_PATCHES, D_MODEL), lambda b: (b, 0, 0)),
        out_shape=jax.ShapeDtypeStruct((BATCH, NUM_PATCHES, D_MODEL), x.dtype),
    )(x, pos_table)
